# Initial kernel scaffold; baseline (speedup 1.0000x reference)
import jax
import jax.numpy as jnp
from jax.experimental import pallas as pl


def kernel(input_ids, word_table, pos_table, ln_gamma, ln_beta):
    raise NotImplementedError("write your pallas kernel here")


# speedup vs baseline: 1.4944x; 1.4944x over previous
import functools

import jax
import jax.numpy as jnp
from jax import lax
from jax.experimental import pallas as pl
from jax.experimental.pallas import tpu as pltpu
from jax.experimental.pallas import tpu_sc as plsc

_B = 4096
_S = 200
_D = 128
_T = _B * _S
_NC, _NS = 2, 16
_NW = _NC * _NS
_IDX_ROWS = _T // 128
_ROWS_PER_W = _IDX_ROWS // _NW
_K = 4
_STEPS = _ROWS_PER_W // _K


def _sc_gather(table, ids2d):
    @functools.partial(
        pl.kernel,
        out_type=jax.ShapeDtypeStruct((_T, _D), jnp.float32),
        mesh=plsc.VectorSubcoreMesh(core_axis_name="c", subcore_axis_name="s"),
        scratch_types=[
            pltpu.VMEM((_K, 128), jnp.int32),
            pltpu.VMEM((_K * 128, _D), jnp.float32),
            pltpu.SemaphoreType.DMA,
        ],
    )
    def k(table_hbm, idx_hbm, out_hbm, idx_v, rows_v, sem):
        wid = lax.axis_index("s") * _NC + lax.axis_index("c")
        row0 = wid * _ROWS_PER_W

        def body(it, carry):
            r = row0 + it * _K
            pltpu.sync_copy(idx_hbm.at[pl.ds(r, _K)], idx_v)
            cps = [
                pltpu.async_copy(
                    table_hbm.at[idx_v.at[j]],
                    rows_v.at[pl.ds(j * 128, 128)],
                    sem,
                )
                for j in range(_K)
            ]
            for c in cps:
                c.wait()
            pltpu.sync_copy(rows_v, out_hbm.at[pl.ds(r * 128, _K * 128)])
            return carry

        lax.fori_loop(0, _STEPS, body, 0)

    return k(table, ids2d)


def _ln_body(x_ref, pos_ref, g_ref, b_ref, o_ref):
    x = x_ref[...] + pos_ref[...]
    mean = jnp.mean(x, axis=-1, keepdims=True)
    xc = x - mean
    var = jnp.mean(xc * xc, axis=-1, keepdims=True)
    inv = lax.rsqrt(var + 1e-12)
    o_ref[...] = xc * inv * g_ref[...] + b_ref[...]


def _tc_layernorm(word3, pos3, g2, b2):
    bb = 8
    return pl.pallas_call(
        _ln_body,
        grid=(_B // bb,),
        in_specs=[
            pl.BlockSpec((bb, _S, _D), lambda i: (i, 0, 0)),
            pl.BlockSpec((1, _S, _D), lambda i: (0, 0, 0)),
            pl.BlockSpec((1, _D), lambda i: (0, 0)),
            pl.BlockSpec((1, _D), lambda i: (0, 0)),
        ],
        out_specs=pl.BlockSpec((bb, _S, _D), lambda i: (i, 0, 0)),
        out_shape=jax.ShapeDtypeStruct((_B, _S, _D), jnp.float32),
    )(word3, pos3, g2, b2)


def kernel(input_ids, word_table, pos_table, ln_gamma, ln_beta):
    ids2d = input_ids.astype(jnp.int32).reshape(_IDX_ROWS, 128)
    word = _sc_gather(word_table, ids2d)
    pos3 = pos_table[:_S].reshape(1, _S, _D)
    return _tc_layernorm(
        word.reshape(_B, _S, _D), pos3,
        ln_gamma.reshape(1, _D), ln_beta.reshape(1, _D),
    )
